# Initial kernel scaffold; baseline (speedup 1.0000x reference)
import jax
import jax.numpy as jnp
from jax.experimental import pallas as pl


def kernel(patch_embeddings):
    raise NotImplementedError("write your pallas kernel here")


# speedup vs baseline: 1.0758x; 1.0758x over previous
import functools

import jax
import jax.numpy as jnp
import numpy as np
from jax import lax
from jax.experimental import pallas as pl
from jax.experimental.pallas import tpu as pltpu
from jax.experimental.pallas import tpu_sc as plsc

_MASK_PCT = 0.75
_B = 64
_NP = 1024
_D = 192
_N_MASKED = int(_MASK_PCT * _NP)
_N_UNMASKED = _NP - _N_MASKED
_ROWS = _B * _N_UNMASKED
_NW = 32
_ROWS_PER_TILE = _ROWS // _NW
_CHUNK = 128
_NCHUNKS = _ROWS_PER_TILE // _CHUNK


def _threefry2x32(k0, k1, x0, x1):
    rot = ((13, 15, 26, 6), (17, 29, 16, 24))
    ks = [np.uint32(k0), np.uint32(k1),
          np.uint32(0x1BD11BDA) ^ np.uint32(k0) ^ np.uint32(k1)]
    x0 = (np.asarray(x0, np.uint32) + ks[0]).astype(np.uint32)
    x1 = (np.asarray(x1, np.uint32) + ks[1]).astype(np.uint32)
    for i in range(5):
        for r in rot[i % 2]:
            x0 = (x0 + x1).astype(np.uint32)
            x1 = ((x1 << np.uint32(r)) | (x1 >> np.uint32(32 - r))).astype(np.uint32)
            x1 = (x1 ^ x0).astype(np.uint32)
        x0 = (x0 + ks[(i + 1) % 3]).astype(np.uint32)
        x1 = (x1 + ks[(i + 2) % 3] + np.uint32(i + 1)).astype(np.uint32)
    return x0, x1


@functools.lru_cache(maxsize=1)
def _mask_constants():
    k0, k1 = _threefry2x32(0, 0, 0, 1)
    i = np.arange(_B * _NP, dtype=np.uint64)
    b0, b1 = _threefry2x32(k0, k1,
                           (i >> np.uint64(32)).astype(np.uint32),
                           (i & np.uint64(0xFFFFFFFF)).astype(np.uint32))
    bits = (b0 ^ b1).reshape(_B, _NP)
    scores = (((bits >> np.uint32(9)) | np.uint32(0x3F800000)).view(np.float32)
              - np.float32(1.0))
    perm = np.argsort(scores, axis=1, kind="stable")
    masked = np.sort(perm[:, :_N_MASKED], axis=1).astype(np.int32)
    unmasked = np.sort(perm[:, _N_MASKED:], axis=1).astype(np.int32)
    bool_mask = np.ones((_B, _NP), dtype=bool)
    np.put_along_axis(bool_mask, unmasked, False, axis=1)
    flat_idx = (np.arange(_B, dtype=np.int64)[:, None] * _NP + unmasked)
    flat_idx = flat_idx.reshape(_NW * _NCHUNKS, _CHUNK).astype(np.int32)
    return masked, unmasked, bool_mask, flat_idx


_CONSTS = _mask_constants()

_sc_mesh = plsc.VectorSubcoreMesh(core_axis_name="c", subcore_axis_name="s")


@functools.partial(
    pl.kernel,
    mesh=_sc_mesh,
    compiler_params=pltpu.CompilerParams(use_tc_tiling_on_sc=False),
    out_type=jax.ShapeDtypeStruct((_ROWS, _D), jnp.float32),
    scratch_types=[
        pltpu.VMEM((_NCHUNKS, _CHUNK), jnp.int32),
        pltpu.VMEM((_ROWS_PER_TILE, _D), jnp.float32),
        pltpu.SemaphoreType.DMA,
    ],
)
def _sc_gather(emb_hbm, idx_hbm, out_hbm, idx_v, rows_v, sem):
    wid = lax.axis_index("s") * 2 + lax.axis_index("c")
    pltpu.sync_copy(idx_hbm.at[pl.ds(wid * _NCHUNKS, _NCHUNKS)], idx_v)
    copies = []
    for j in range(_NCHUNKS):
        copies.append(
            pltpu.async_copy(
                emb_hbm.at[idx_v.at[j]],
                rows_v.at[pl.ds(j * _CHUNK, _CHUNK)],
                sem,
            )
        )
    for c in copies:
        c.wait()
    pltpu.sync_copy(rows_v, out_hbm.at[pl.ds(wid * _ROWS_PER_TILE, _ROWS_PER_TILE)])


def kernel(patch_embeddings):
    masked, unmasked, bool_mask, flat_idx = _CONSTS
    emb = patch_embeddings.reshape(_B * _NP, _D)
    out = _sc_gather(emb, jnp.asarray(flat_idx))
    unmasked_patches = out.reshape(_B, _N_UNMASKED, _D)
    return (
        unmasked_patches,
        jnp.asarray(bool_mask),
        jnp.asarray(masked),
        jnp.asarray(unmasked),
    )


# speedup vs baseline: 3.2302x; 3.0028x over previous
import functools

import jax
import jax.numpy as jnp
import numpy as np
from jax import lax
from jax.experimental import pallas as pl
from jax.experimental.pallas import tpu as pltpu
from jax.experimental.pallas import tpu_sc as plsc

_MASK_PCT = 0.75
_B = 64
_NP = 1024
_D = 192
_N_MASKED = int(_MASK_PCT * _NP)
_N_UNMASKED = _NP - _N_MASKED
_NW = 32
_ROWS = _B * _D
_ROWS_PER_TILE = _ROWS // _NW
_RCHUNK = 32
_NCHUNK = _D // _RCHUNK
_LANES = 16


def _threefry2x32(k0, k1, x0, x1):
    rot = ((13, 15, 26, 6), (17, 29, 16, 24))
    ks = [np.uint32(k0), np.uint32(k1),
          np.uint32(0x1BD11BDA) ^ np.uint32(k0) ^ np.uint32(k1)]
    x0 = (np.asarray(x0, np.uint32) + ks[0]).astype(np.uint32)
    x1 = (np.asarray(x1, np.uint32) + ks[1]).astype(np.uint32)
    for i in range(5):
        for r in rot[i % 2]:
            x0 = (x0 + x1).astype(np.uint32)
            x1 = ((x1 << np.uint32(r)) | (x1 >> np.uint32(32 - r))).astype(np.uint32)
            x1 = (x1 ^ x0).astype(np.uint32)
        x0 = (x0 + ks[(i + 1) % 3]).astype(np.uint32)
        x1 = (x1 + ks[(i + 2) % 3] + np.uint32(i + 1)).astype(np.uint32)
    return x0, x1


@functools.lru_cache(maxsize=1)
def _mask_constants():
    k0, k1 = _threefry2x32(0, 0, 0, 1)
    i = np.arange(_B * _NP, dtype=np.uint64)
    b0, b1 = _threefry2x32(k0, k1,
                           (i >> np.uint64(32)).astype(np.uint32),
                           (i & np.uint64(0xFFFFFFFF)).astype(np.uint32))
    bits = (b0 ^ b1).reshape(_B, _NP)
    scores = (((bits >> np.uint32(9)) | np.uint32(0x3F800000)).view(np.float32)
              - np.float32(1.0))
    perm = np.argsort(scores, axis=1, kind="stable")
    masked = np.sort(perm[:, :_N_MASKED], axis=1).astype(np.int32)
    unmasked = np.sort(perm[:, _N_MASKED:], axis=1).astype(np.int32)
    bool_mask = np.ones((_B, _NP), dtype=bool)
    np.put_along_axis(bool_mask, unmasked, False, axis=1)
    return masked, unmasked, bool_mask


_CONSTS = _mask_constants()

_sc_mesh = plsc.VectorSubcoreMesh(core_axis_name="c", subcore_axis_name="s")


@functools.partial(
    pl.kernel,
    mesh=_sc_mesh,
    compiler_params=pltpu.CompilerParams(needs_layout_passes=False),
    out_type=jax.ShapeDtypeStruct((_ROWS, _N_UNMASKED), jnp.float32),
    scratch_types=[
        pltpu.VMEM((2, _RCHUNK, _NP), jnp.float32),
        pltpu.VMEM((2, _RCHUNK, _N_UNMASKED), jnp.float32),
        pltpu.VMEM((2 * _N_UNMASKED,), jnp.int32),
        pltpu.SemaphoreType.DMA,
        pltpu.SemaphoreType.DMA,
    ],
)
def _sc_compact(emb_hbm, idx_hbm, out_hbm, in_v, out_v, idx_v, in_sem, out_sem):
    wid = lax.axis_index("s") * 2 + lax.axis_index("c")
    row0 = wid * _ROWS_PER_TILE
    pltpu.sync_copy(idx_hbm.at[pl.ds(wid * (2 * _N_UNMASKED), 2 * _N_UNMASKED)],
                    idx_v)

    def start_in(gkk, buf):
        pltpu.async_copy(emb_hbm.at[pl.ds(row0 + gkk * _RCHUNK, _RCHUNK)],
                         in_v.at[buf], in_sem)

    def wait_in(buf):
        pltpu.make_async_copy(emb_hbm.at[pl.ds(0, _RCHUNK)], in_v.at[buf],
                              in_sem).wait()

    def start_out(gkk, buf):
        pltpu.async_copy(out_v.at[buf],
                         out_hbm.at[pl.ds(row0 + gkk * _RCHUNK, _RCHUNK)],
                         out_sem)

    def wait_out(buf):
        pltpu.make_async_copy(out_v.at[buf], out_hbm.at[pl.ds(0, _RCHUNK)],
                              out_sem).wait()

    start_in(0, 0)
    total = 2 * _NCHUNK

    def process(gkk, bloc, buf):
        wait_in(buf)
        @pl.when(gkk >= 2)
        def _():
            wait_out(buf)
        for j in range(_N_UNMASKED // _LANES):
            idx16 = idx_v[pl.ds(bloc * _N_UNMASKED + j * _LANES, _LANES)]

            def row_body(r, rvec):
                out_v[buf, r, pl.ds(j * _LANES, _LANES)] = plsc.load_gather(
                    in_v.at[buf], [rvec, idx16])
                return rvec + 1

            lax.fori_loop(0, _RCHUNK, row_body,
                          jnp.zeros((_LANES,), jnp.int32), unroll=False)
        start_out(gkk, buf)

    for bloc in (0, 1):
        def pair_body(p, c, bloc=bloc):
            for parity in (0, 1):
                gkk = bloc * _NCHUNK + 2 * p + parity
                if bloc == 0:
                    start_in(gkk + 1, 1 - parity)
                else:
                    @pl.when(gkk + 1 < total)
                    def _():
                        start_in(gkk + 1, 1 - parity)
                process(gkk, bloc, parity)
            return c

        lax.fori_loop(0, _NCHUNK // 2, pair_body, 0, unroll=False)
    wait_out(0)
    wait_out(1)


def kernel(patch_embeddings):
    masked, unmasked, bool_mask = _CONSTS
    emb_t = jnp.transpose(patch_embeddings, (0, 2, 1)).reshape(_ROWS, _NP)
    out = _sc_compact(emb_t, jnp.asarray(unmasked.reshape(-1)))
    unmasked_patches = jnp.transpose(
        out.reshape(_B, _D, _N_UNMASKED), (0, 2, 1))
    return (
        unmasked_patches,
        jnp.asarray(bool_mask),
        jnp.asarray(masked),
        jnp.asarray(unmasked),
    )


# speedup vs baseline: 3.5172x; 1.0888x over previous
import functools

import jax
import jax.numpy as jnp
import numpy as np
from jax import lax
from jax.experimental import pallas as pl
from jax.experimental.pallas import tpu as pltpu
from jax.experimental.pallas import tpu_sc as plsc

_MASK_PCT = 0.75
_B = 64
_NP = 1024
_D = 192
_N_MASKED = int(_MASK_PCT * _NP)
_N_UNMASKED = _NP - _N_MASKED
_NW = 32
_ROWS = _B * _D
_ROWS_PER_TILE = _ROWS // _NW
_RCHUNK = 32
_NCHUNK = _D // _RCHUNK
_LANES = 16


def _threefry2x32(k0, k1, x0, x1):
    rot = ((13, 15, 26, 6), (17, 29, 16, 24))
    ks = [np.uint32(k0), np.uint32(k1),
          np.uint32(0x1BD11BDA) ^ np.uint32(k0) ^ np.uint32(k1)]
    x0 = (np.asarray(x0, np.uint32) + ks[0]).astype(np.uint32)
    x1 = (np.asarray(x1, np.uint32) + ks[1]).astype(np.uint32)
    for i in range(5):
        for r in rot[i % 2]:
            x0 = (x0 + x1).astype(np.uint32)
            x1 = ((x1 << np.uint32(r)) | (x1 >> np.uint32(32 - r))).astype(np.uint32)
            x1 = (x1 ^ x0).astype(np.uint32)
        x0 = (x0 + ks[(i + 1) % 3]).astype(np.uint32)
        x1 = (x1 + ks[(i + 2) % 3] + np.uint32(i + 1)).astype(np.uint32)
    return x0, x1


@functools.lru_cache(maxsize=1)
def _mask_constants():
    k0, k1 = _threefry2x32(0, 0, 0, 1)
    i = np.arange(_B * _NP, dtype=np.uint64)
    b0, b1 = _threefry2x32(k0, k1,
                           (i >> np.uint64(32)).astype(np.uint32),
                           (i & np.uint64(0xFFFFFFFF)).astype(np.uint32))
    bits = (b0 ^ b1).reshape(_B, _NP)
    scores = (((bits >> np.uint32(9)) | np.uint32(0x3F800000)).view(np.float32)
              - np.float32(1.0))
    perm = np.argsort(scores, axis=1, kind="stable")
    masked = np.sort(perm[:, :_N_MASKED], axis=1).astype(np.int32)
    unmasked = np.sort(perm[:, _N_MASKED:], axis=1).astype(np.int32)
    bool_mask = np.ones((_B, _NP), dtype=bool)
    np.put_along_axis(bool_mask, unmasked, False, axis=1)
    return masked, unmasked, bool_mask


_CONSTS = _mask_constants()

_sc_mesh = plsc.VectorSubcoreMesh(core_axis_name="c", subcore_axis_name="s")


@functools.partial(
    pl.kernel,
    mesh=_sc_mesh,
    compiler_params=pltpu.CompilerParams(needs_layout_passes=False),
    out_type=jax.ShapeDtypeStruct((_ROWS, _N_UNMASKED), jnp.float32),
    scratch_types=[
        pltpu.VMEM((2, _RCHUNK, _NP), jnp.float32),
        pltpu.VMEM((2, _RCHUNK, _N_UNMASKED), jnp.float32),
        pltpu.VMEM((2 * _N_UNMASKED,), jnp.int32),
        pltpu.SemaphoreType.DMA,
        pltpu.SemaphoreType.DMA,
    ],
)
def _sc_compact(emb_hbm, idx_hbm, out_hbm, in_v, out_v, idx_v, in_sem, out_sem):
    wid = lax.axis_index("s") * 2 + lax.axis_index("c")
    row0 = wid * _ROWS_PER_TILE
    pltpu.sync_copy(idx_hbm.at[pl.ds(wid * (2 * _N_UNMASKED), 2 * _N_UNMASKED)],
                    idx_v)

    def start_in(gkk, buf):
        pltpu.async_copy(emb_hbm.at[pl.ds(row0 + gkk * _RCHUNK, _RCHUNK)],
                         in_v.at[buf], in_sem)

    def wait_in(buf):
        pltpu.make_async_copy(emb_hbm.at[pl.ds(0, _RCHUNK)], in_v.at[buf],
                              in_sem).wait()

    def start_out(gkk, buf):
        pltpu.async_copy(out_v.at[buf],
                         out_hbm.at[pl.ds(row0 + gkk * _RCHUNK, _RCHUNK)],
                         out_sem)

    def wait_out(buf):
        pltpu.make_async_copy(out_v.at[buf], out_hbm.at[pl.ds(0, _RCHUNK)],
                              out_sem).wait()

    start_in(0, 0)
    total = 2 * _NCHUNK

    def process(gkk, bloc, buf):
        wait_in(buf)
        @pl.when(gkk >= 2)
        def _():
            wait_out(buf)
        idxs = tuple(
            idx_v[pl.ds(bloc * _N_UNMASKED + j * _LANES, _LANES)]
            for j in range(_N_UNMASKED // _LANES))

        def row_body(r, rvec):
            for j in range(_N_UNMASKED // _LANES):
                out_v[buf, r, pl.ds(j * _LANES, _LANES)] = plsc.load_gather(
                    in_v.at[buf], [rvec, idxs[j]])
            return rvec + 1

        lax.fori_loop(0, _RCHUNK, row_body,
                      jnp.zeros((_LANES,), jnp.int32), unroll=False)
        start_out(gkk, buf)

    for bloc in (0, 1):
        def pair_body(p, c, bloc=bloc):
            for parity in (0, 1):
                gkk = bloc * _NCHUNK + 2 * p + parity
                if bloc == 0:
                    start_in(gkk + 1, 1 - parity)
                else:
                    @pl.when(gkk + 1 < total)
                    def _():
                        start_in(gkk + 1, 1 - parity)
                process(gkk, bloc, parity)
            return c

        lax.fori_loop(0, _NCHUNK // 2, pair_body, 0, unroll=False)
    wait_out(0)
    wait_out(1)


def kernel(patch_embeddings):
    masked, unmasked, bool_mask = _CONSTS
    emb_t = jnp.transpose(patch_embeddings, (0, 2, 1)).reshape(_ROWS, _NP)
    out = _sc_compact(emb_t, jnp.asarray(unmasked.reshape(-1)))
    unmasked_patches = jnp.transpose(
        out.reshape(_B, _D, _N_UNMASKED), (0, 2, 1))
    return (
        unmasked_patches,
        jnp.asarray(bool_mask),
        jnp.asarray(masked),
        jnp.asarray(unmasked),
    )


# speedup vs baseline: 3.5493x; 1.0091x over previous
import functools

import jax
import jax.numpy as jnp
import numpy as np
from jax import lax
from jax.experimental import pallas as pl
from jax.experimental.pallas import tpu as pltpu
from jax.experimental.pallas import tpu_sc as plsc

_MASK_PCT = 0.75
_B = 64
_NP = 1024
_D = 192
_N_MASKED = int(_MASK_PCT * _NP)
_N_UNMASKED = _NP - _N_MASKED
_NW = 32
_ROWS = _B * _D
_ROWS_PER_TILE = _ROWS // _NW
_RCHUNK = 32
_NCHUNK = _D // _RCHUNK
_LANES = 16


def _threefry2x32(k0, k1, x0, x1):
    rot = ((13, 15, 26, 6), (17, 29, 16, 24))
    ks = [np.uint32(k0), np.uint32(k1),
          np.uint32(0x1BD11BDA) ^ np.uint32(k0) ^ np.uint32(k1)]
    x0 = (np.asarray(x0, np.uint32) + ks[0]).astype(np.uint32)
    x1 = (np.asarray(x1, np.uint32) + ks[1]).astype(np.uint32)
    for i in range(5):
        for r in rot[i % 2]:
            x0 = (x0 + x1).astype(np.uint32)
            x1 = ((x1 << np.uint32(r)) | (x1 >> np.uint32(32 - r))).astype(np.uint32)
            x1 = (x1 ^ x0).astype(np.uint32)
        x0 = (x0 + ks[(i + 1) % 3]).astype(np.uint32)
        x1 = (x1 + ks[(i + 2) % 3] + np.uint32(i + 1)).astype(np.uint32)
    return x0, x1


@functools.lru_cache(maxsize=1)
def _mask_constants():
    k0, k1 = _threefry2x32(0, 0, 0, 1)
    i = np.arange(_B * _NP, dtype=np.uint64)
    b0, b1 = _threefry2x32(k0, k1,
                           (i >> np.uint64(32)).astype(np.uint32),
                           (i & np.uint64(0xFFFFFFFF)).astype(np.uint32))
    bits = (b0 ^ b1).reshape(_B, _NP)
    scores = (((bits >> np.uint32(9)) | np.uint32(0x3F800000)).view(np.float32)
              - np.float32(1.0))
    perm = np.argsort(scores, axis=1, kind="stable")
    masked = np.sort(perm[:, :_N_MASKED], axis=1).astype(np.int32)
    unmasked = np.sort(perm[:, _N_MASKED:], axis=1).astype(np.int32)
    bool_mask = np.ones((_B, _NP), dtype=bool)
    np.put_along_axis(bool_mask, unmasked, False, axis=1)
    return masked, unmasked, bool_mask


_CONSTS = _mask_constants()

_sc_mesh = plsc.VectorSubcoreMesh(core_axis_name="c", subcore_axis_name="s")


@functools.partial(
    pl.kernel,
    mesh=_sc_mesh,
    compiler_params=pltpu.CompilerParams(needs_layout_passes=False),
    out_type=jax.ShapeDtypeStruct((_ROWS, _N_UNMASKED), jnp.float32),
    scratch_types=[
        pltpu.VMEM((2, _RCHUNK, _NP), jnp.float32),
        pltpu.VMEM((2, _RCHUNK, _N_UNMASKED), jnp.float32),
        pltpu.VMEM((2 * _N_UNMASKED,), jnp.int32),
        pltpu.SemaphoreType.DMA,
        pltpu.SemaphoreType.DMA,
    ],
)
def _sc_compact(emb_hbm, idx_hbm, out_hbm, in_v, out_v, idx_v, in_sem, out_sem):
    wid = lax.axis_index("s") * 2 + lax.axis_index("c")
    row0 = wid * _ROWS_PER_TILE
    pltpu.sync_copy(idx_hbm.at[pl.ds(wid * (2 * _N_UNMASKED), 2 * _N_UNMASKED)],
                    idx_v)

    def start_in(gkk, buf):
        pltpu.async_copy(emb_hbm.at[pl.ds(row0 + gkk * _RCHUNK, _RCHUNK)],
                         in_v.at[buf], in_sem)

    def wait_in(buf):
        pltpu.make_async_copy(emb_hbm.at[pl.ds(0, _RCHUNK)], in_v.at[buf],
                              in_sem).wait()

    def start_out(gkk, buf):
        pltpu.async_copy(out_v.at[buf],
                         out_hbm.at[pl.ds(row0 + gkk * _RCHUNK, _RCHUNK)],
                         out_sem)

    def wait_out(buf):
        pltpu.make_async_copy(out_v.at[buf], out_hbm.at[pl.ds(0, _RCHUNK)],
                              out_sem).wait()

    start_in(0, 0)
    total = 2 * _NCHUNK

    def pair_body(p, c):
        for parity in (0, 1):
            gkk = 2 * p + parity

            @pl.when(gkk + 1 < total)
            def _():
                start_in(gkk + 1, 1 - parity)

            wait_in(parity)
            @pl.when(gkk >= 2)
            def _():
                wait_out(parity)
            ioff = jnp.where(gkk >= _NCHUNK, _N_UNMASKED, 0)
            idxs = tuple(
                idx_v[pl.ds(ioff + j * _LANES, _LANES)]
                for j in range(_N_UNMASKED // _LANES))

            def row_body(r, rvec):
                for j in range(_N_UNMASKED // _LANES):
                    out_v[parity, r, pl.ds(j * _LANES, _LANES)] = \
                        plsc.load_gather(in_v.at[parity], [rvec, idxs[j]])
                return rvec + 1

            lax.fori_loop(0, _RCHUNK, row_body,
                          jnp.zeros((_LANES,), jnp.int32), unroll=False)
            start_out(gkk, parity)
        return c

    lax.fori_loop(0, _NCHUNK, pair_body, 0, unroll=False)
    wait_out(0)
    wait_out(1)


def kernel(patch_embeddings):
    masked, unmasked, bool_mask = _CONSTS
    emb_t = jnp.transpose(patch_embeddings, (0, 2, 1)).reshape(_ROWS, _NP)
    out = _sc_compact(emb_t, jnp.asarray(unmasked.reshape(-1)))
    unmasked_patches = jnp.transpose(
        out.reshape(_B, _D, _N_UNMASKED), (0, 2, 1))
    return (
        unmasked_patches,
        jnp.asarray(bool_mask),
        jnp.asarray(masked),
        jnp.asarray(unmasked),
    )


# speedup vs baseline: 3.5860x; 1.0103x over previous
import functools

import jax
import jax.numpy as jnp
import numpy as np
from jax import lax
from jax.experimental import pallas as pl
from jax.experimental.pallas import tpu as pltpu
from jax.experimental.pallas import tpu_sc as plsc

_MASK_PCT = 0.75
_B = 64
_NP = 1024
_D = 192
_N_MASKED = int(_MASK_PCT * _NP)
_N_UNMASKED = _NP - _N_MASKED
_NW = 32
_ROWS = _B * _D
_ROWS_PER_TILE = _ROWS // _NW
_RCHUNK = 32
_NCHUNK = _D // _RCHUNK
_LANES = 16


def _threefry2x32(k0, k1, x0, x1):
    rot = ((13, 15, 26, 6), (17, 29, 16, 24))
    ks = [np.uint32(k0), np.uint32(k1),
          np.uint32(0x1BD11BDA) ^ np.uint32(k0) ^ np.uint32(k1)]
    x0 = (np.asarray(x0, np.uint32) + ks[0]).astype(np.uint32)
    x1 = (np.asarray(x1, np.uint32) + ks[1]).astype(np.uint32)
    for i in range(5):
        for r in rot[i % 2]:
            x0 = (x0 + x1).astype(np.uint32)
            x1 = ((x1 << np.uint32(r)) | (x1 >> np.uint32(32 - r))).astype(np.uint32)
            x1 = (x1 ^ x0).astype(np.uint32)
        x0 = (x0 + ks[(i + 1) % 3]).astype(np.uint32)
        x1 = (x1 + ks[(i + 2) % 3] + np.uint32(i + 1)).astype(np.uint32)
    return x0, x1


@functools.lru_cache(maxsize=1)
def _mask_constants():
    k0, k1 = _threefry2x32(0, 0, 0, 1)
    i = np.arange(_B * _NP, dtype=np.uint64)
    b0, b1 = _threefry2x32(k0, k1,
                           (i >> np.uint64(32)).astype(np.uint32),
                           (i & np.uint64(0xFFFFFFFF)).astype(np.uint32))
    bits = (b0 ^ b1).reshape(_B, _NP)
    scores = (((bits >> np.uint32(9)) | np.uint32(0x3F800000)).view(np.float32)
              - np.float32(1.0))
    perm = np.argsort(scores, axis=1, kind="stable")
    masked = np.sort(perm[:, :_N_MASKED], axis=1).astype(np.int32)
    unmasked = np.sort(perm[:, _N_MASKED:], axis=1).astype(np.int32)
    bool_mask = np.ones((_B, _NP), dtype=bool)
    np.put_along_axis(bool_mask, unmasked, False, axis=1)
    return masked, unmasked, bool_mask


_CONSTS = _mask_constants()

_sc_mesh = plsc.VectorSubcoreMesh(core_axis_name="c", subcore_axis_name="s")


@functools.partial(
    pl.kernel,
    mesh=_sc_mesh,
    compiler_params=pltpu.CompilerParams(needs_layout_passes=False),
    out_type=jax.ShapeDtypeStruct((_ROWS, _N_UNMASKED), jnp.float32),
    scratch_types=[
        pltpu.VMEM((3, _RCHUNK, _NP), jnp.float32),
        pltpu.VMEM((3, _RCHUNK, _N_UNMASKED), jnp.float32),
        pltpu.VMEM((2 * _N_UNMASKED,), jnp.int32),
        pltpu.SemaphoreType.DMA,
        pltpu.SemaphoreType.DMA,
    ],
)
def _sc_compact(emb_hbm, idx_hbm, out_hbm, in_v, out_v, idx_v, in_sem, out_sem):
    wid = lax.axis_index("s") * 2 + lax.axis_index("c")
    row0 = wid * _ROWS_PER_TILE
    pltpu.sync_copy(idx_hbm.at[pl.ds(wid * (2 * _N_UNMASKED), 2 * _N_UNMASKED)],
                    idx_v)

    def start_in(gkk, buf):
        pltpu.async_copy(emb_hbm.at[pl.ds(row0 + gkk * _RCHUNK, _RCHUNK)],
                         in_v.at[buf], in_sem)

    def wait_in(buf):
        pltpu.make_async_copy(emb_hbm.at[pl.ds(0, _RCHUNK)], in_v.at[buf],
                              in_sem).wait()

    def start_out(gkk, buf):
        pltpu.async_copy(out_v.at[buf],
                         out_hbm.at[pl.ds(row0 + gkk * _RCHUNK, _RCHUNK)],
                         out_sem)

    def wait_out(buf):
        pltpu.make_async_copy(out_v.at[buf], out_hbm.at[pl.ds(0, _RCHUNK)],
                              out_sem).wait()

    start_in(0, 0)
    start_in(1, 1)
    total = 2 * _NCHUNK

    def triple_body(p, c):
        for q in (0, 1, 2):
            gkk = 3 * p + q

            @pl.when(gkk + 2 < total)
            def _():
                start_in(gkk + 2, (q + 2) % 3)

            wait_in(q)
            @pl.when(gkk >= 3)
            def _():
                wait_out(q)
            ioff = jnp.where(gkk >= _NCHUNK, _N_UNMASKED, 0)
            idxs = tuple(
                idx_v[pl.ds(ioff + j * _LANES, _LANES)]
                for j in range(_N_UNMASKED // _LANES))

            def row_body(r, rvec):
                for j in range(_N_UNMASKED // _LANES):
                    out_v[q, r, pl.ds(j * _LANES, _LANES)] = \
                        plsc.load_gather(in_v.at[q], [rvec, idxs[j]])
                return rvec + 1

            lax.fori_loop(0, _RCHUNK, row_body,
                          jnp.zeros((_LANES,), jnp.int32), unroll=False)
            start_out(gkk, q)
        return c

    lax.fori_loop(0, total // 3, triple_body, 0, unroll=False)
    wait_out(0)
    wait_out(1)
    wait_out(2)


def kernel(patch_embeddings):
    masked, unmasked, bool_mask = _CONSTS
    emb_t = jnp.transpose(patch_embeddings, (0, 2, 1)).reshape(_ROWS, _NP)
    out = _sc_compact(emb_t, jnp.asarray(unmasked.reshape(-1)))
    unmasked_patches = jnp.transpose(
        out.reshape(_B, _D, _N_UNMASKED), (0, 2, 1))
    return (
        unmasked_patches,
        jnp.asarray(bool_mask),
        jnp.asarray(masked),
        jnp.asarray(unmasked),
    )
